# Initial kernel scaffold; baseline (speedup 1.0000x reference)
import jax
import jax.numpy as jnp
from jax.experimental import pallas as pl


def kernel(x0, x1, edge_index0, edge_index1, rank_mapping0, rank_mapping1, params):
    raise NotImplementedError("write your pallas kernel here")


# speedup vs baseline: 6.0275x; 6.0275x over previous
import functools

import jax
import jax.numpy as jnp
from jax import lax
from jax.experimental import pallas as pl
from jax.experimental.pallas import tpu as pltpu
from jax.experimental.pallas import tpu_sc as plsc

N = 10000
E2 = 160000
D = 128
NS = 16
CHUNK = 128
NCHUNK = 79
EPT = NCHUNK * CHUNK
PE = NS * EPT
NPAD = 10240
DEN_R = NPAD // D
ROWS_PER_TILE = NPAD // NS


def _sc1_run_rel(spf, srcI, dstI, numH, denH,
                 sv, den_v, src_buf, dst_buf, nm_buf, ridx, den_sp):
    t = lax.axis_index("s")
    ebase = t * EPT
    zero16 = jnp.zeros((16,), jnp.float32)
    iota16 = lax.iota(jnp.int32, 16)

    pltpu.sync_copy(spf, sv)

    @pl.loop(0, DEN_R)
    def _(r):
        for q in range(8):
            den_v[r, pl.ds(q * 16, 16)] = zero16

    @pl.when(t < 10)
    def _():
        pltpu.sync_copy(den_v.at[pl.ds(0, 8), :], den_sp.at[pl.ds(t * 8, 8), :])
    plsc.subcore_barrier()

    @pl.loop(0, NCHUNK)
    def _(k):
        off = ebase + k * CHUNK
        pltpu.sync_copy(srcI.at[pl.ds(off, CHUNK)], src_buf)
        pltpu.sync_copy(dstI.at[pl.ds(off, CHUNK)], dst_buf)
        for v in range(8):
            sidx = src_buf[pl.ds(v * 16, 16)]
            didx = dst_buf[pl.ds(v * 16, 16)]
            a = plsc.load_gather(sv, [didx * 2])
            b = plsc.load_gather(sv, [sidx * 2 + 1])
            x = a + b
            x = jnp.where(x >= 0.0, x, x * 0.2)
            nm = jnp.exp(x)
            pos = off + v * 16 + iota16
            nm = jnp.where(pos < E2, nm, 0.0)
            nm_buf[pl.ds(v * 16, 16)] = nm
            plsc.addupdate_scatter(
                den_v,
                [lax.shift_right_logical(didx, 7), lax.bitwise_and(didx, 127)],
                nm)
        pltpu.sync_copy(nm_buf, numH.at[pl.ds(off, CHUNK)])

    for v in range(5):
        ridx[pl.ds(v * 16, 16)] = v * 16 + iota16
    pltpu.sync_copy(den_v, den_sp.at[ridx], add=True)
    plsc.subcore_barrier()

    @pl.when(t < 10)
    def _():
        pltpu.sync_copy(den_sp.at[pl.ds(t * 8, 8), :], denH.at[pl.ds(t * 8, 8), :])


def _sc1_body(sp0f, sp1f, src0, dst0, src1, dst1, num0, num1, den0, den1,
              sv, den_v, src_buf, dst_buf, nm_buf, ridx, den_sp):
    c = lax.axis_index("c")

    @pl.when(c == 0)
    def _():
        _sc1_run_rel(sp0f, src0, dst0, num0, den0,
                     sv, den_v, src_buf, dst_buf, nm_buf, ridx, den_sp)

    @pl.when(c == 1)
    def _():
        _sc1_run_rel(sp1f, src1, dst1, num1, den1,
                     sv, den_v, src_buf, dst_buf, nm_buf, ridx, den_sp)


@functools.lru_cache(maxsize=None)
def _make_sc1(interpret=False, mesh=None):
    if mesh is None:
        mesh = plsc.VectorSubcoreMesh(core_axis_name="c",
                                      subcore_axis_name="s",
                                      num_cores=2, num_subcores=NS)
    return pl.kernel(
        _sc1_body,
        out_type=(jax.ShapeDtypeStruct((PE,), jnp.float32),
                  jax.ShapeDtypeStruct((PE,), jnp.float32),
                  jax.ShapeDtypeStruct((DEN_R, D), jnp.float32),
                  jax.ShapeDtypeStruct((DEN_R, D), jnp.float32)),
        mesh=mesh,
        interpret=interpret,
        compiler_params=pltpu.CompilerParams(needs_layout_passes=False),
        scratch_types=[
            pltpu.VMEM((2 * N,), jnp.float32),
            pltpu.VMEM((DEN_R, D), jnp.float32),
            pltpu.VMEM((CHUNK,), jnp.int32),
            pltpu.VMEM((CHUNK,), jnp.int32),
            pltpu.VMEM((CHUNK,), jnp.float32),
            pltpu.VMEM((DEN_R,), jnp.int32),
            pltpu.VMEM_SHARED((DEN_R, D), jnp.float32),
        ],
    )


def _sc2_run_rel(g, srcI, dstI, numI, outH,
                 src_buf, dst_buf, w_buf, rows, out_sp, sem):
    t = lax.axis_index("s")
    ebase = t * EPT
    zero16 = jnp.zeros((16,), jnp.float32)

    @pl.loop(0, CHUNK)
    def _(r):
        for q in range(8):
            rows[r, pl.ds(q * 16, 16)] = zero16
    for q in range(5):
        pltpu.sync_copy(rows, out_sp.at[pl.ds(t * ROWS_PER_TILE + q * CHUNK, CHUNK), :])
    plsc.subcore_barrier()

    @pl.loop(0, NCHUNK)
    def _(k):
        off = ebase + k * CHUNK
        pltpu.sync_copy(srcI.at[pl.ds(off, CHUNK)], src_buf)
        pltpu.sync_copy(dstI.at[pl.ds(off, CHUNK)], dst_buf)
        pltpu.sync_copy(numI.at[pl.ds(off, CHUNK)], w_buf)
        pltpu.async_copy(g.at[src_buf, :], rows, sem).wait()

        @pl.loop(0, CHUNK)
        def _(e):
            wb = plsc.load_gather(w_buf, [jnp.zeros((16,), jnp.int32) + e])
            for q in range(8):
                rows[e, pl.ds(q * 16, 16)] = rows[e, pl.ds(q * 16, 16)] * wb

        pltpu.sync_copy(rows, out_sp.at[dst_buf], add=True)

    plsc.subcore_barrier()
    pltpu.sync_copy(out_sp.at[pl.ds(t * ROWS_PER_TILE, ROWS_PER_TILE), :],
                    outH.at[pl.ds(t * ROWS_PER_TILE, ROWS_PER_TILE), :])


def _sc2_body(g0, g1, num0, num1, src0, dst0, src1, dst1, gat_out,
              src_buf, dst_buf, w_buf, rows, out_sp, sem):
    c = lax.axis_index("c")

    @pl.when(c == 0)
    def _():
        _sc2_run_rel(g0, src0, dst0, num0, gat_out.at[0],
                     src_buf, dst_buf, w_buf, rows, out_sp, sem)

    @pl.when(c == 1)
    def _():
        _sc2_run_rel(g1, src1, dst1, num1, gat_out.at[1],
                     src_buf, dst_buf, w_buf, rows, out_sp, sem)


@functools.lru_cache(maxsize=None)
def _make_sc2(interpret=False, mesh=None):
    if mesh is None:
        mesh = plsc.VectorSubcoreMesh(core_axis_name="c",
                                      subcore_axis_name="s",
                                      num_cores=2, num_subcores=NS)
    return pl.kernel(
        _sc2_body,
        out_type=jax.ShapeDtypeStruct((2, NPAD, D), jnp.float32),
        mesh=mesh,
        interpret=interpret,
        compiler_params=pltpu.CompilerParams(needs_layout_passes=False),
        scratch_types=[
            pltpu.VMEM((CHUNK,), jnp.int32),
            pltpu.VMEM((CHUNK,), jnp.int32),
            pltpu.VMEM((CHUNK,), jnp.float32),
            pltpu.VMEM((CHUNK, D), jnp.float32),
            pltpu.VMEM_SHARED((NPAD, D), jnp.float32),
            pltpu.SemaphoreType.DMA,
        ],
    )


def _sc1_call(*args):
    return _make_sc1()(*args)


def _sc2_call(*args):
    return _make_sc2()(*args)


def _dot(a, b):
    return lax.dot_general(a, b, (((1,), (0,)), ((), ())),
                           preferred_element_type=jnp.float32)


def _tc_pre_body(x0_r, x1_r, W0_r, W1_r, A0_r, A1_r, Wr0_r, Wr1_r, Wsk_r, B_r,
                 hs0_r, hs1_r, sp0_r, sp1_r, pre0_r, pre1_r):
    x0 = x0_r[...]
    x1 = x1_r[...]
    W0 = W0_r[...]
    W1 = W1_r[...]
    hd0 = _dot(x1, W0)
    hs0 = _dot(x0, W0)
    hd1 = _dot(x0, W1)
    hs1 = _dot(x1, W1)
    a1_0 = A0_r[0:128, :]
    a2_0 = A0_r[128:256, :]
    a1_1 = A1_r[0:128, :]
    a2_1 = A1_r[128:256, :]
    sp0_r[...] = (jnp.concatenate([_dot(hd0, a1_0), _dot(hs0, a2_0)], axis=1)
                  + B_r[3:4, 0:2])
    sp1_r[...] = (jnp.concatenate([_dot(hd1, a1_1), _dot(hs1, a2_1)], axis=1)
                  + B_r[4:5, 0:2])
    hs0_r[...] = hs0
    hs1_r[...] = hs1
    pre0_r[...] = (_dot(x0, Wsk_r[...]) + B_r[2:3, :]
                   + _dot(x0, Wr1_r[...]) + B_r[1:2, :])
    pre1_r[...] = (_dot(x1, Wsk_r[...]) + B_r[2:3, :]
                   + _dot(x1, Wr0_r[...]) + B_r[0:1, :])


def _tc_pre(x0, x1, W0, W1, A0, A1, Wr0, Wr1, Wsk, B):
    R = 1000
    full = lambda s: pl.BlockSpec(s, lambda i: (0, 0))
    rowb = lambda s: pl.BlockSpec(s, lambda i: (i, 0))
    return pl.pallas_call(
        _tc_pre_body,
        grid=(N // R,),
        in_specs=[rowb((R, D)), rowb((R, D)), full((D, D)), full((D, D)),
                  full((256, 1)), full((256, 1)), full((D, D)), full((D, D)),
                  full((D, D)), full((8, D))],
        out_specs=[rowb((R, D)), rowb((R, D)), rowb((R, 2)), rowb((R, 2)),
                   rowb((R, D)), rowb((R, D))],
        out_shape=[jax.ShapeDtypeStruct((N, D), jnp.float32),
                   jax.ShapeDtypeStruct((N, D), jnp.float32),
                   jax.ShapeDtypeStruct((N, 2), jnp.float32),
                   jax.ShapeDtypeStruct((N, 2), jnp.float32),
                   jax.ShapeDtypeStruct((N, D), jnp.float32),
                   jax.ShapeDtypeStruct((N, D), jnp.float32)],
    )(x0, x1, W0, W1, A0, A1, Wr0, Wr1, Wsk, B)


def _tc_g_body(hs0_r, hs1_r, d0_r, d1_r, g0_r, g1_r):
    g0_r[...] = hs0_r[...] * (1.0 / (d0_r[...] + 1e-16))
    g1_r[...] = hs1_r[...] * (1.0 / (d1_r[...] + 1e-16))


def _tc_g(hs0, hs1, d0, d1):
    R = 1000
    rowb = lambda s: pl.BlockSpec(s, lambda i: (i, 0))
    return pl.pallas_call(
        _tc_g_body,
        grid=(N // R,),
        in_specs=[rowb((R, D)), rowb((R, D)), rowb((R, 1)), rowb((R, 1))],
        out_specs=[rowb((R, D)), rowb((R, D))],
        out_shape=[jax.ShapeDtypeStruct((N, D), jnp.float32),
                   jax.ShapeDtypeStruct((N, D), jnp.float32)],
    )(hs0, hs1, d0, d1)


def _bn_relu(t, g, b):
    mu = jnp.mean(t, axis=0, keepdims=True)
    var = jnp.mean((t - mu) ** 2, axis=0, keepdims=True)
    return jnp.maximum(g * (t - mu) / jnp.sqrt(var + 1e-5) + b, 0.0)


def _tc_post_body(pre_r, g_r, Gb_r, out_r):
    t = pre_r[0] + g_r[0]
    out_r[...] = _bn_relu(t, Gb_r[0:1, :], Gb_r[1:2, :])[None]


def _tc_post(pre, g, Gb):
    return pl.pallas_call(
        _tc_post_body,
        grid=(2,),
        in_specs=[pl.BlockSpec((1, N, D), lambda i: (i, 0, 0)),
                  pl.BlockSpec((1, N, D), lambda i: (1 - i, 0, 0)),
                  pl.BlockSpec((2, D), lambda i: (0, 0))],
        out_specs=pl.BlockSpec((1, N, D), lambda i: (i, 0, 0)),
        out_shape=jax.ShapeDtypeStruct((2, N, D), jnp.float32),
    )(pre, g, Gb)


def _tc_final_body(pre_r, g_r, Gb_r, W1_r, W2_r, Mb_r, out_r):
    t = pre_r[0] + g_r[0]
    y = _bn_relu(t, Gb_r[0:1, :], Gb_r[1:2, :])
    h = _dot(y, W1_r[...]) + Mb_r[0:1, :]
    h = _bn_relu(h, Mb_r[1:2, :], Mb_r[2:3, :])
    out_r[...] = (_dot(h, W2_r[...]) + Mb_r[3:4, :])[None]


def _tc_final(pre, g, Gb, W1, W2, Mb):
    return pl.pallas_call(
        _tc_final_body,
        grid=(2,),
        in_specs=[pl.BlockSpec((1, N, D), lambda i: (i, 0, 0)),
                  pl.BlockSpec((1, N, D), lambda i: (1 - i, 0, 0)),
                  pl.BlockSpec((2, D), lambda i: (0, 0)),
                  pl.BlockSpec((D, D), lambda i: (0, 0)),
                  pl.BlockSpec((D, D), lambda i: (0, 0)),
                  pl.BlockSpec((8, D), lambda i: (0, 0))],
        out_specs=pl.BlockSpec((1, N, D), lambda i: (i, 0, 0)),
        out_shape=jax.ShapeDtypeStruct((2, N, D), jnp.float32),
    )(pre, g, Gb, W1, W2, Mb)


def _prep_idx(ei):
    pad = jnp.zeros((PE - E2,), jnp.int32)
    return (jnp.concatenate([ei[:, 0, :].reshape(-1), pad]),
            jnp.concatenate([ei[:, 1, :].reshape(-1), pad]))


def _layer_weights(params, l):
    g0 = params['gat'][l][0]
    g1 = params['gat'][l][1]
    sk = params['skip'][l]
    B = jnp.zeros((8, D), jnp.float32)
    B = B.at[0].set(g0['bias']).at[1].set(g1['bias']).at[2].set(sk['b'])
    B = B.at[3, 0].set(g0['a_b'][0]).at[4, 0].set(g1['a_b'][0])
    return (g0['W'], g1['W'], g0['a_w'], g1['a_w'],
            g0['W_res'], g1['W_res'], sk['W'], B)


def kernel(x0, x1, edge_index0, edge_index1, rank_mapping0, rank_mapping1,
           params):
    del rank_mapping0, rank_mapping1
    src0, dst0 = _prep_idx(edge_index0)
    src1, dst1 = _prep_idx(edge_index1)

    xs = (x0, x1)
    for l in range(2):
        W0, W1, A0, A1, Wr0, Wr1, Wsk, B = _layer_weights(params, l)
        hs0, hs1, sp0, sp1, pre0, pre1 = _tc_pre(xs[0], xs[1], W0, W1, A0, A1,
                                                 Wr0, Wr1, Wsk, B)
        num0, num1, den0, den1 = _sc1_call(sp0.reshape(-1), sp1.reshape(-1),
                                           src0, dst0, src1, dst1)
        d0 = den0.reshape(NPAD)[:N].reshape(N, 1)
        d1 = den1.reshape(NPAD)[:N].reshape(N, 1)
        g0, g1 = _tc_g(hs0, hs1, d0, d1)
        gat = _sc2_call(g0, g1, num0, num1, src0, dst0, src1, dst1)[:, :N, :]
        pre = jnp.stack([pre0, pre1])
        bnp = params['bn'][l]
        Gb = jnp.stack([bnp['g'], bnp['b']])
        if l == 0:
            out2 = _tc_post(pre, gat, Gb)
            xs = (out2[0], out2[1])
        else:
            m = params['mlp']
            Mb = jnp.zeros((8, D), jnp.float32)
            Mb = Mb.at[0].set(m['b1']).at[1].set(m['g']).at[2].set(m['beta'])
            Mb = Mb.at[3].set(m['b2'])
            final = _tc_final(pre, gat, Gb, m['W1'], m['W2'], Mb)
    return final.reshape(2 * N, D)


# speedup vs baseline: 6.9724x; 1.1568x over previous
import functools

import jax
import jax.numpy as jnp
from jax import lax
from jax.experimental import pallas as pl
from jax.experimental.pallas import tpu as pltpu
from jax.experimental.pallas import tpu_sc as plsc

N = 10000
E2 = 160000
D = 128
NS = 16
CHUNK = 128
NCHUNK = 80
EPT = NCHUNK * CHUNK
PE = NS * EPT
NPAD = 10240
DEN_R = NPAD // D
ROWS_PER_TILE = NPAD // NS


def _sc1_run_rel(spf, srcI, dstI, numH, denH,
                 sv, den_v, src_buf, dst_buf, nm_buf, ridx, den_sp):
    t = lax.axis_index("s")
    ebase = t * EPT
    zero16 = jnp.zeros((16,), jnp.float32)
    iota16 = lax.iota(jnp.int32, 16)

    pltpu.sync_copy(spf, sv)

    @pl.loop(0, DEN_R)
    def _(r):
        for q in range(8):
            den_v[r, pl.ds(q * 16, 16)] = zero16

    @pl.when(t < 10)
    def _():
        pltpu.sync_copy(den_v.at[pl.ds(0, 8), :], den_sp.at[pl.ds(t * 8, 8), :])
    plsc.subcore_barrier()

    @pl.loop(0, NCHUNK)
    def _(k):
        off = ebase + k * CHUNK
        pltpu.sync_copy(srcI.at[pl.ds(off, CHUNK)], src_buf)
        pltpu.sync_copy(dstI.at[pl.ds(off, CHUNK)], dst_buf)
        for v in range(8):
            sidx = src_buf[pl.ds(v * 16, 16)]
            didx = dst_buf[pl.ds(v * 16, 16)]
            a = plsc.load_gather(sv, [didx * 2])
            b = plsc.load_gather(sv, [sidx * 2 + 1])
            x = a + b
            x = jnp.where(x >= 0.0, x, x * 0.2)
            nm = jnp.exp(x)
            pos = off + v * 16 + iota16
            nm = jnp.where(pos < E2, nm, 0.0)
            nm_buf[pl.ds(v * 16, 16)] = nm
            plsc.addupdate_scatter(
                den_v,
                [lax.shift_right_logical(didx, 7), lax.bitwise_and(didx, 127)],
                nm)
        pltpu.sync_copy(nm_buf, numH.at[pl.ds(off, CHUNK)])

    for v in range(5):
        ridx[pl.ds(v * 16, 16)] = v * 16 + iota16
    pltpu.sync_copy(den_v, den_sp.at[ridx], add=True)
    plsc.subcore_barrier()

    @pl.when(t < 10)
    def _():
        pltpu.sync_copy(den_sp.at[pl.ds(t * 8, 8), :], denH.at[pl.ds(t * 8, 8), :])


def _sc1_body(sp0f, sp1f, src0, dst0, src1, dst1, num0, num1, den0, den1,
              sv, den_v, src_buf, dst_buf, nm_buf, ridx, den_sp):
    c = lax.axis_index("c")

    @pl.when(c == 0)
    def _():
        _sc1_run_rel(sp0f, src0, dst0, num0, den0,
                     sv, den_v, src_buf, dst_buf, nm_buf, ridx, den_sp)

    @pl.when(c == 1)
    def _():
        _sc1_run_rel(sp1f, src1, dst1, num1, den1,
                     sv, den_v, src_buf, dst_buf, nm_buf, ridx, den_sp)


@functools.lru_cache(maxsize=None)
def _make_sc1(interpret=False, mesh=None):
    if mesh is None:
        mesh = plsc.VectorSubcoreMesh(core_axis_name="c",
                                      subcore_axis_name="s",
                                      num_cores=2, num_subcores=NS)
    return pl.kernel(
        _sc1_body,
        out_type=(jax.ShapeDtypeStruct((PE,), jnp.float32),
                  jax.ShapeDtypeStruct((PE,), jnp.float32),
                  jax.ShapeDtypeStruct((DEN_R, D), jnp.float32),
                  jax.ShapeDtypeStruct((DEN_R, D), jnp.float32)),
        mesh=mesh,
        interpret=interpret,
        compiler_params=pltpu.CompilerParams(needs_layout_passes=False),
        scratch_types=[
            pltpu.VMEM((2 * N,), jnp.float32),
            pltpu.VMEM((DEN_R, D), jnp.float32),
            pltpu.VMEM((CHUNK,), jnp.int32),
            pltpu.VMEM((CHUNK,), jnp.int32),
            pltpu.VMEM((CHUNK,), jnp.float32),
            pltpu.VMEM((DEN_R,), jnp.int32),
            pltpu.VMEM_SHARED((DEN_R, D), jnp.float32),
        ],
    )


def _sc2_run_rel(g, sdat, numR, outH,
                 sd0, sd1, w0, w1, rows0, rows1,
                 out_sp, gsem0, gsem1, ssem0, ssem1):
    t = lax.axis_index("s")
    base = t * NCHUNK
    zero16 = jnp.zeros((16,), jnp.float32)

    def fetch(k, sd, w, rows, gsem):
        row = base + k
        pltpu.sync_copy(sdat.at[row], sd)
        pltpu.sync_copy(numR.at[row], w)
        pltpu.async_copy(g.at[sd.at[0], :], rows, gsem)

    def process(sd, w, rows, gsem, ssem):
        pltpu.make_async_copy(g.at[sd.at[0], :], rows, gsem).wait()

        @pl.loop(0, CHUNK, unroll=2)
        def _(e):
            wb = plsc.load_gather(w, [jnp.zeros((16,), jnp.int32) + e])
            for q in range(8):
                rows[e, pl.ds(q * 16, 16)] = rows[e, pl.ds(q * 16, 16)] * wb

        pltpu.async_copy(rows, out_sp.at[sd.at[1]], ssem, add=True)

    def wait_scatter(sd, rows, ssem):
        pltpu.make_async_copy(rows, out_sp.at[sd.at[1]], ssem).wait()

    @pl.loop(0, CHUNK)
    def _(r):
        for q in range(8):
            rows0[r, pl.ds(q * 16, 16)] = zero16

    @pl.when(t < 15)
    def _():
        for q in range(4):
            pltpu.sync_copy(rows0, out_sp.at[pl.ds(t * 632 + q * CHUNK, CHUNK), :])
        pltpu.sync_copy(rows0.at[pl.ds(0, 120), :],
                        out_sp.at[pl.ds(t * 632 + 512, 120), :])

    @pl.when(t == 15)
    def _():
        for q in range(4):
            pltpu.sync_copy(rows0, out_sp.at[pl.ds(9480 + q * CHUNK, CHUNK), :])
        pltpu.sync_copy(rows0.at[pl.ds(0, 8), :], out_sp.at[pl.ds(9992, 8), :])
    plsc.subcore_barrier()

    fetch(0, sd0, w0, rows0, gsem0)
    fetch(1, sd1, w1, rows1, gsem1)
    process(sd0, w0, rows0, gsem0, ssem0)

    @pl.loop(0, (NCHUNK - 2) // 2)
    def _(j):
        wait_scatter(sd0, rows0, ssem0)
        fetch(2 * j + 2, sd0, w0, rows0, gsem0)
        process(sd1, w1, rows1, gsem1, ssem1)
        wait_scatter(sd1, rows1, ssem1)
        fetch(2 * j + 3, sd1, w1, rows1, gsem1)
        process(sd0, w0, rows0, gsem0, ssem0)

    wait_scatter(sd0, rows0, ssem0)
    process(sd1, w1, rows1, gsem1, ssem1)
    wait_scatter(sd1, rows1, ssem1)

    plsc.subcore_barrier()

    @pl.when(t < 15)
    def _():
        pltpu.sync_copy(out_sp.at[pl.ds(t * 632, 632), :],
                        outH.at[pl.ds(t * 632, 632), :])

    @pl.when(t == 15)
    def _():
        pltpu.sync_copy(out_sp.at[pl.ds(9480, 520), :],
                        outH.at[pl.ds(9480, 520), :])


def _sc2_body(g0, g1, sdat0, sdat1, numR0, numR1, gat_out,
              sd0, sd1, w0, w1, rows0, rows1,
              out_sp, gsem0, gsem1, ssem0, ssem1):
    c = lax.axis_index("c")

    @pl.when(c == 0)
    def _():
        _sc2_run_rel(g0, sdat0, numR0, gat_out.at[0],
                     sd0, sd1, w0, w1, rows0, rows1,
                     out_sp, gsem0, gsem1, ssem0, ssem1)

    @pl.when(c == 1)
    def _():
        _sc2_run_rel(g1, sdat1, numR1, gat_out.at[1],
                     sd0, sd1, w0, w1, rows0, rows1,
                     out_sp, gsem0, gsem1, ssem0, ssem1)


@functools.lru_cache(maxsize=None)
def _make_sc2(interpret=False, mesh=None):
    if mesh is None:
        mesh = plsc.VectorSubcoreMesh(core_axis_name="c",
                                      subcore_axis_name="s",
                                      num_cores=2, num_subcores=NS)
    return pl.kernel(
        _sc2_body,
        out_type=jax.ShapeDtypeStruct((2, N, D), jnp.float32),
        mesh=mesh,
        interpret=interpret,
        compiler_params=pltpu.CompilerParams(needs_layout_passes=False),
        scratch_types=[
            pltpu.VMEM((2, CHUNK), jnp.int32),
            pltpu.VMEM((2, CHUNK), jnp.int32),
            pltpu.VMEM((CHUNK,), jnp.float32),
            pltpu.VMEM((CHUNK,), jnp.float32),
            pltpu.VMEM((CHUNK, D), jnp.float32),
            pltpu.VMEM((CHUNK, D), jnp.float32),
            pltpu.VMEM_SHARED((N, D), jnp.float32),
            pltpu.SemaphoreType.DMA,
            pltpu.SemaphoreType.DMA,
            pltpu.SemaphoreType.DMA,
            pltpu.SemaphoreType.DMA,
        ],
    )


def _sc1_call(*args):
    return _make_sc1()(*args)


def _sc2_call(*args):
    return _make_sc2()(*args)


def _dot(a, b):
    return lax.dot_general(a, b, (((1,), (0,)), ((), ())),
                           preferred_element_type=jnp.float32)


def _tc_pre_body(x0_r, x1_r, W0_r, W1_r, A0_r, A1_r, Wr0_r, Wr1_r, Wsk_r, B_r,
                 hs0_r, hs1_r, sp0_r, sp1_r, pre0_r, pre1_r):
    x0 = x0_r[...]
    x1 = x1_r[...]
    W0 = W0_r[...]
    W1 = W1_r[...]
    hd0 = _dot(x1, W0)
    hs0 = _dot(x0, W0)
    hd1 = _dot(x0, W1)
    hs1 = _dot(x1, W1)
    a1_0 = A0_r[0:128, :]
    a2_0 = A0_r[128:256, :]
    a1_1 = A1_r[0:128, :]
    a2_1 = A1_r[128:256, :]
    sp0_r[...] = (jnp.concatenate([_dot(hd0, a1_0), _dot(hs0, a2_0)], axis=1)
                  + B_r[3:4, 0:2])
    sp1_r[...] = (jnp.concatenate([_dot(hd1, a1_1), _dot(hs1, a2_1)], axis=1)
                  + B_r[4:5, 0:2])
    hs0_r[...] = hs0
    hs1_r[...] = hs1
    pre0_r[...] = (_dot(x0, Wsk_r[...]) + B_r[2:3, :]
                   + _dot(x0, Wr1_r[...]) + B_r[1:2, :])
    pre1_r[...] = (_dot(x1, Wsk_r[...]) + B_r[2:3, :]
                   + _dot(x1, Wr0_r[...]) + B_r[0:1, :])


def _tc_pre(x0, x1, W0, W1, A0, A1, Wr0, Wr1, Wsk, B):
    R = 1000
    full = lambda s: pl.BlockSpec(s, lambda i: (0, 0))
    rowb = lambda s: pl.BlockSpec(s, lambda i: (i, 0))
    return pl.pallas_call(
        _tc_pre_body,
        grid=(N // R,),
        in_specs=[rowb((R, D)), rowb((R, D)), full((D, D)), full((D, D)),
                  full((256, 1)), full((256, 1)), full((D, D)), full((D, D)),
                  full((D, D)), full((8, D))],
        out_specs=[rowb((R, D)), rowb((R, D)), rowb((R, 2)), rowb((R, 2)),
                   rowb((R, D)), rowb((R, D))],
        out_shape=[jax.ShapeDtypeStruct((N, D), jnp.float32),
                   jax.ShapeDtypeStruct((N, D), jnp.float32),
                   jax.ShapeDtypeStruct((N, 2), jnp.float32),
                   jax.ShapeDtypeStruct((N, 2), jnp.float32),
                   jax.ShapeDtypeStruct((N, D), jnp.float32),
                   jax.ShapeDtypeStruct((N, D), jnp.float32)],
    )(x0, x1, W0, W1, A0, A1, Wr0, Wr1, Wsk, B)


def _tc_g_body(hs0_r, hs1_r, d0_r, d1_r, g0_r, g1_r):
    g0_r[...] = hs0_r[...] * (1.0 / (d0_r[...] + 1e-16))
    g1_r[...] = hs1_r[...] * (1.0 / (d1_r[...] + 1e-16))


def _tc_g(hs0, hs1, d0, d1):
    R = 1000
    rowb = lambda s: pl.BlockSpec(s, lambda i: (i, 0))
    return pl.pallas_call(
        _tc_g_body,
        grid=(N // R,),
        in_specs=[rowb((R, D)), rowb((R, D)), rowb((R, 1)), rowb((R, 1))],
        out_specs=[rowb((R, D)), rowb((R, D))],
        out_shape=[jax.ShapeDtypeStruct((N, D), jnp.float32),
                   jax.ShapeDtypeStruct((N, D), jnp.float32)],
    )(hs0, hs1, d0, d1)


def _bn_relu(t, g, b):
    mu = jnp.mean(t, axis=0, keepdims=True)
    var = jnp.mean((t - mu) ** 2, axis=0, keepdims=True)
    return jnp.maximum(g * (t - mu) / jnp.sqrt(var + 1e-5) + b, 0.0)


def _tc_post_body(pre_r, g_r, Gb_r, out_r):
    t = pre_r[0] + g_r[0]
    out_r[...] = _bn_relu(t, Gb_r[0:1, :], Gb_r[1:2, :])[None]


def _tc_post(pre, g, Gb):
    return pl.pallas_call(
        _tc_post_body,
        grid=(2,),
        in_specs=[pl.BlockSpec((1, N, D), lambda i: (i, 0, 0)),
                  pl.BlockSpec((1, N, D), lambda i: (1 - i, 0, 0)),
                  pl.BlockSpec((2, D), lambda i: (0, 0))],
        out_specs=pl.BlockSpec((1, N, D), lambda i: (i, 0, 0)),
        out_shape=jax.ShapeDtypeStruct((2, N, D), jnp.float32),
    )(pre, g, Gb)


def _tc_final_body(pre_r, g_r, Gb_r, W1_r, W2_r, Mb_r, out_r):
    t = pre_r[0] + g_r[0]
    y = _bn_relu(t, Gb_r[0:1, :], Gb_r[1:2, :])
    h = _dot(y, W1_r[...]) + Mb_r[0:1, :]
    h = _bn_relu(h, Mb_r[1:2, :], Mb_r[2:3, :])
    out_r[...] = (_dot(h, W2_r[...]) + Mb_r[3:4, :])[None]


def _tc_final(pre, g, Gb, W1, W2, Mb):
    return pl.pallas_call(
        _tc_final_body,
        grid=(2,),
        in_specs=[pl.BlockSpec((1, N, D), lambda i: (i, 0, 0)),
                  pl.BlockSpec((1, N, D), lambda i: (1 - i, 0, 0)),
                  pl.BlockSpec((2, D), lambda i: (0, 0)),
                  pl.BlockSpec((D, D), lambda i: (0, 0)),
                  pl.BlockSpec((D, D), lambda i: (0, 0)),
                  pl.BlockSpec((8, D), lambda i: (0, 0))],
        out_specs=pl.BlockSpec((1, N, D), lambda i: (i, 0, 0)),
        out_shape=jax.ShapeDtypeStruct((2, N, D), jnp.float32),
    )(pre, g, Gb, W1, W2, Mb)


def _prep_idx(ei):
    pad = jnp.zeros((PE - E2,), jnp.int32)
    return (jnp.concatenate([ei[:, 0, :].reshape(-1), pad]),
            jnp.concatenate([ei[:, 1, :].reshape(-1), pad]))


def _layer_weights(params, l):
    g0 = params['gat'][l][0]
    g1 = params['gat'][l][1]
    sk = params['skip'][l]
    B = jnp.zeros((8, D), jnp.float32)
    B = B.at[0].set(g0['bias']).at[1].set(g1['bias']).at[2].set(sk['b'])
    B = B.at[3, 0].set(g0['a_b'][0]).at[4, 0].set(g1['a_b'][0])
    return (g0['W'], g1['W'], g0['a_w'], g1['a_w'],
            g0['W_res'], g1['W_res'], sk['W'], B)


def kernel(x0, x1, edge_index0, edge_index1, rank_mapping0, rank_mapping1,
           params):
    del rank_mapping0, rank_mapping1
    src0, dst0 = _prep_idx(edge_index0)
    src1, dst1 = _prep_idx(edge_index1)
    sdat0 = jnp.stack([src0.reshape(NS * NCHUNK, CHUNK),
                       dst0.reshape(NS * NCHUNK, CHUNK)], axis=1)
    sdat1 = jnp.stack([src1.reshape(NS * NCHUNK, CHUNK),
                       dst1.reshape(NS * NCHUNK, CHUNK)], axis=1)

    xs = (x0, x1)
    for l in range(2):
        W0, W1, A0, A1, Wr0, Wr1, Wsk, B = _layer_weights(params, l)
        hs0, hs1, sp0, sp1, pre0, pre1 = _tc_pre(xs[0], xs[1], W0, W1, A0, A1,
                                                 Wr0, Wr1, Wsk, B)
        num0, num1, den0, den1 = _sc1_call(sp0.reshape(-1), sp1.reshape(-1),
                                           src0, dst0, src1, dst1)
        numR0 = num0.reshape(NS * NCHUNK, CHUNK)
        numR1 = num1.reshape(NS * NCHUNK, CHUNK)
        d0 = den0.reshape(NPAD)[:N].reshape(N, 1)
        d1 = den1.reshape(NPAD)[:N].reshape(N, 1)
        g0, g1 = _tc_g(hs0, hs1, d0, d1)
        gat = _sc2_call(g0, g1, sdat0, sdat1, numR0, numR1)
        pre = jnp.stack([pre0, pre1])
        bnp = params['bn'][l]
        Gb = jnp.stack([bnp['g'], bnp['b']])
        if l == 0:
            out2 = _tc_post(pre, gat, Gb)
            xs = (out2[0], out2[1])
        else:
            m = params['mlp']
            Mb = jnp.zeros((8, D), jnp.float32)
            Mb = Mb.at[0].set(m['b1']).at[1].set(m['g']).at[2].set(m['beta'])
            Mb = Mb.at[3].set(m['b2'])
            final = _tc_final(pre, gat, Gb, m['W1'], m['W2'], Mb)
    return final.reshape(2 * N, D)
